# baseline (speedup 1.0000x reference)
import jax
import jax.numpy as jnp
import numpy as np
from jax.experimental import pallas as pl
from jax.experimental.pallas import tpu as pltpu

_B, _T, _D = 256, 512, 256
_BB = 8


def _attn_kernel(ip_ref, wt_ref, b_ref, ctx_ref, out_ref):
    w = wt_ref[...]
    bias = b_ref[...]
    ctx = ctx_ref[...]
    for i in range(_BB):
        x = ip_ref[i]
        h = jnp.tanh(
            jnp.dot(
                x.astype(jnp.bfloat16), w,
                preferred_element_type=jnp.float32,
            )
            + bias
        )
        s = jnp.sum(h * ctx, axis=-1, keepdims=True)
        e = jnp.exp2(s)
        denom = jnp.sum(e, axis=0, keepdims=True)
        acc = jnp.sum(e * x, axis=0, keepdims=True)
        out_ref[i : i + 1, :] = acc * (1.0 / denom)


def kernel(ip, W, b, context):
    wt = W.T.astype(jnp.bfloat16)
    b2 = b.reshape(1, _D)
    ctx2 = context.reshape(1, _D) * np.float32(np.log2(np.e))
    return pl.pallas_call(
        _attn_kernel,
        grid=(_B // _BB,),
        in_specs=[
            pl.BlockSpec((_BB, _T, _D), lambda i: (i, 0, 0)),
            pl.BlockSpec((_D, _D), lambda i: (0, 0)),
            pl.BlockSpec((1, _D), lambda i: (0, 0)),
            pl.BlockSpec((1, _D), lambda i: (0, 0)),
        ],
        out_specs=pl.BlockSpec((_BB, _D), lambda i: (i, 0)),
        out_shape=jax.ShapeDtypeStruct((_B, _D), jnp.float32),
        compiler_params=pltpu.CompilerParams(
            dimension_semantics=("parallel",),
        ),
    )(ip, wt, b2, ctx2)


# speedup vs baseline: 1.2217x; 1.2217x over previous
import jax
import jax.numpy as jnp
import numpy as np
from jax.experimental import pallas as pl
from jax.experimental.pallas import tpu as pltpu

_B, _T, _D = 256, 512, 256
_BB = 32


def _attn_kernel(ip_ref, wt_ref, b_ref, ctx_ref, out_ref):
    w = wt_ref[...]
    bias = b_ref[...]
    ctx = ctx_ref[...]
    for i in range(_BB):
        x = ip_ref[i]
        h = jnp.tanh(
            jnp.dot(
                x.astype(jnp.bfloat16), w,
                preferred_element_type=jnp.float32,
            )
            + bias
        )
        s = jnp.sum(h * ctx, axis=-1, keepdims=True)
        e = jnp.exp2(s)
        denom = jnp.sum(e, axis=0, keepdims=True)
        acc = jnp.sum(e * x, axis=0, keepdims=True)
        out_ref[i : i + 1, :] = acc * (1.0 / denom)


def kernel(ip, W, b, context):
    wt = W.T.astype(jnp.bfloat16)
    b2 = b.reshape(1, _D)
    ctx2 = context.reshape(1, _D) * np.float32(np.log2(np.e))
    return pl.pallas_call(
        _attn_kernel,
        grid=(_B // _BB,),
        in_specs=[
            pl.BlockSpec((_BB, _T, _D), lambda i: (i, 0, 0)),
            pl.BlockSpec((_D, _D), lambda i: (0, 0)),
            pl.BlockSpec((1, _D), lambda i: (0, 0)),
            pl.BlockSpec((1, _D), lambda i: (0, 0)),
        ],
        out_specs=pl.BlockSpec((_BB, _D), lambda i: (i, 0)),
        out_shape=jax.ShapeDtypeStruct((_B, _D), jnp.float32),
        compiler_params=pltpu.CompilerParams(
            dimension_semantics=("parallel",),
        ),
    )(ip, wt, b2, ctx2)


# speedup vs baseline: 1.2244x; 1.0022x over previous
import jax
import jax.numpy as jnp
import numpy as np
from jax.experimental import pallas as pl
from jax.experimental.pallas import tpu as pltpu

_B, _T, _D = 256, 512, 256
_BB = 32


def _attn_kernel(ip_ref, wt_ref, b_ref, ctx_ref, out_ref):
    w = wt_ref[...]
    bias = b_ref[...]
    ctx = ctx_ref[...]
    for i in range(_BB):
        x = ip_ref[i]
        h = jnp.tanh(
            jnp.dot(
                x.astype(jnp.bfloat16), w,
                preferred_element_type=jnp.float32,
            )
            + bias
        )
        s = jnp.sum(h * ctx, axis=-1, keepdims=True)
        e = jnp.exp2(s)
        denom = jnp.sum(e, axis=0, keepdims=True)
        acc = jnp.sum(e * x, axis=0, keepdims=True)
        out_ref[i : i + 1, :] = acc * (1.0 / denom)


def kernel(ip, W, b, context):
    wt = W.T.astype(jnp.bfloat16)
    b2 = b.reshape(1, _D)
    ctx2 = context.reshape(1, _D) * np.float32(np.log2(np.e))
    return pl.pallas_call(
        _attn_kernel,
        grid=(_B // _BB,),
        in_specs=[
            pl.BlockSpec((_BB, _T, _D), lambda i: (i, 0, 0)),
            pl.BlockSpec((_D, _D), lambda i: (0, 0)),
            pl.BlockSpec((1, _D), lambda i: (0, 0)),
            pl.BlockSpec((1, _D), lambda i: (0, 0)),
        ],
        out_specs=pl.BlockSpec((_BB, _D), lambda i: (i, 0)),
        out_shape=jax.ShapeDtypeStruct((_B, _D), jnp.float32),
        compiler_params=pltpu.CompilerParams(
            dimension_semantics=("parallel",),
            vmem_limit_bytes=56 * 1024 * 1024,
        ),
    )(ip, wt, b2, ctx2)
